# Initial kernel scaffold; baseline (speedup 1.0000x reference)
import jax
import jax.numpy as jnp
from jax.experimental import pallas as pl


def kernel(in_feat, edge_index, W1, b1, W2, b2):
    raise NotImplementedError("write your pallas kernel here")


# speedup vs baseline: 5.3945x; 5.3945x over previous
import functools

import jax
import jax.numpy as jnp
from jax import lax
from jax.experimental import pallas as pl
from jax.experimental.pallas import tpu as pltpu
from jax.experimental.pallas import tpu_sc as plsc

N = 10000
E = 320000
D = 128
NC = 2
NS = 16
L = 16
NW = NC * NS
CH = 128
N_CHUNKS = E // CH
N_PAD = 10240
RPT = N_PAD // NS

_mesh = plsc.VectorSubcoreMesh(core_axis_name="c", subcore_axis_name="s",
                               num_cores=NC, num_subcores=NS)


def _worker_chunk_range(wid):
    c0 = (wid * N_CHUNKS) // NW
    c1 = ((wid + 1) * N_CHUNKS) // NW
    return c0, c1


def _fill_rows(ref, value):
    vec = jnp.full((L,), value, dtype=jnp.float32)
    width = ref.shape[-1]

    def body(r, _):
        for j in range(width // L):
            ref[r, pl.ds(j * L, L)] = vec
        return _

    lax.fori_loop(0, ref.shape[0], body, None)


@functools.partial(
    pl.kernel,
    out_type=jax.ShapeDtypeStruct((NC, N_PAD, D), jnp.float32),
    mesh=_mesh,
    scratch_types=[
        pltpu.VMEM((2, CH), jnp.int32),
        pltpu.VMEM((CH, D), jnp.float32),
        pltpu.VMEM_SHARED((N_PAD, D), jnp.float32),
    ],
)
def _deg_kernel(idx_hbm, out_hbm, sidx, cbuf, tab):
    cid = lax.axis_index("c")
    tid = lax.axis_index("s")
    wid = tid * NC + cid
    base = tid * RPT

    _fill_rows(cbuf, 0.0)
    for j in range(RPT // CH):
        pltpu.sync_copy(cbuf, tab.at[pl.ds(base + j * CH, CH)])
    _fill_rows(cbuf, 1.0)
    plsc.subcore_barrier()

    c0, c1 = _worker_chunk_range(wid)

    def body(i, _):
        pltpu.sync_copy(idx_hbm.at[pl.ds(i * CH, CH)], sidx.at[0])
        pltpu.sync_copy(cbuf, tab.at[sidx.at[0]], add=True)
        return _

    lax.fori_loop(c0, c1, body, None)
    plsc.subcore_barrier()

    pltpu.sync_copy(tab.at[pl.ds(base, RPT)],
                    out_hbm.at[cid, pl.ds(base, RPT)])


@functools.partial(
    pl.kernel,
    out_type=jax.ShapeDtypeStruct((NC, N_PAD, D), jnp.float32),
    mesh=_mesh,
    scratch_types=[
        pltpu.VMEM((2, CH), jnp.int32),
        pltpu.VMEM((2, CH), jnp.int32),
        pltpu.VMEM((CH, D), jnp.float32),
        pltpu.VMEM_SHARED((N_PAD, D), jnp.float32),
        pltpu.SemaphoreType.DMA,
    ],
)
def _agg_kernel(y_hbm, src_hbm, dst_hbm, out_hbm, sidx, didx, rows, acc, sem):
    cid = lax.axis_index("c")
    tid = lax.axis_index("s")
    wid = tid * NC + cid
    base = tid * RPT

    _fill_rows(rows, 0.0)
    for j in range(RPT // CH):
        pltpu.sync_copy(rows, acc.at[pl.ds(base + j * CH, CH)])
    rem = RPT % CH
    if rem:
        pltpu.sync_copy(rows.at[pl.ds(0, rem)],
                        acc.at[pl.ds(base + (RPT // CH) * CH, rem)])
    plsc.subcore_barrier()

    c0, c1 = _worker_chunk_range(wid)

    def body(i, _):
        e0 = i * CH
        pltpu.sync_copy(src_hbm.at[pl.ds(e0, CH)], sidx.at[0])
        pltpu.async_copy(y_hbm.at[sidx.at[0]], rows, sem).wait()
        pltpu.sync_copy(dst_hbm.at[pl.ds(e0, CH)], didx.at[0])
        pltpu.sync_copy(rows, acc.at[didx.at[0]], add=True)
        return _

    lax.fori_loop(c0, c1, body, None)
    plsc.subcore_barrier()

    pltpu.sync_copy(acc.at[pl.ds(base, RPT)],
                    out_hbm.at[cid, pl.ds(base, RPT)])


def _norm_from_deg(deg_cols):
    deg = deg_cols[:, 0:1] + deg_cols[:, 1:2]
    return lax.rsqrt(jnp.where(deg > 0, deg, 1.0))


_MB = 2000


def _prescale_body(x_ref, dsrc_ref, o_ref):
    o_ref[...] = x_ref[...] * _norm_from_deg(dsrc_ref[...])


def _prescale(x, dsrc_t):
    grid = N // _MB
    return pl.pallas_call(
        _prescale_body,
        grid=(grid,),
        in_specs=[
            pl.BlockSpec((_MB, D), lambda i: (i, 0)),
            pl.BlockSpec((_MB, 2), lambda i: (i, 0)),
        ],
        out_specs=pl.BlockSpec((_MB, D), lambda i: (i, 0)),
        out_shape=jax.ShapeDtypeStruct((N, D), jnp.float32),
    )(x, dsrc_t)


def _make_mm_body(scale_out):
    def body(p_ref, ddst_ref, dsrc_ref, w_ref, b_ref, o_ref):
        agg = (p_ref[0] + p_ref[1]) * _norm_from_deg(ddst_ref[...])
        h = jnp.dot(agg, w_ref[...], preferred_element_type=jnp.float32)
        h = h + b_ref[...]
        if scale_out:
            h = h * _norm_from_deg(dsrc_ref[...])
        o_ref[...] = h
    return body


def _mm(p, ddst_t, dsrc_t, w, b, scale_out):
    grid = N // _MB
    return pl.pallas_call(
        _make_mm_body(scale_out),
        grid=(grid,),
        in_specs=[
            pl.BlockSpec((NC, _MB, D), lambda i: (0, i, 0)),
            pl.BlockSpec((_MB, 2), lambda i: (i, 0)),
            pl.BlockSpec((_MB, 2), lambda i: (i, 0)),
            pl.BlockSpec((D, D), lambda i: (0, 0)),
            pl.BlockSpec((1, D), lambda i: (0, 0)),
        ],
        out_specs=pl.BlockSpec((_MB, D), lambda i: (i, 0)),
        out_shape=jax.ShapeDtypeStruct((N, D), jnp.float32),
    )(p, ddst_t, dsrc_t, w, b)


def kernel(in_feat, edge_index, W1, b1, W2, b2):
    src = edge_index[0].astype(jnp.int32)
    dst = edge_index[1].astype(jnp.int32)

    dsrc_t = _deg_kernel(src)[:, :N, 0].T
    ddst_t = _deg_kernel(dst)[:, :N, 0].T

    y0 = _prescale(in_feat, dsrc_t)
    p1 = _agg_kernel(y0, src, dst)
    y1 = _mm(p1, ddst_t, dsrc_t, W1, b1.reshape(1, D), scale_out=True)
    p2 = _agg_kernel(y1, src, dst)
    h2 = _mm(p2, ddst_t, dsrc_t, W2, b2.reshape(1, D), scale_out=False)
    return h2
